# Initial kernel scaffold; baseline (speedup 1.0000x reference)
import jax
import jax.numpy as jnp
from jax.experimental import pallas as pl


def kernel(inputs, adj, feat_data, W1, b1, W3, b3):
    raise NotImplementedError("write your pallas kernel here")


# speedup vs baseline: 4.4384x; 4.4384x over previous
import functools

import jax
import jax.numpy as jnp
from jax import lax
from jax.experimental import pallas as pl
from jax.experimental.pallas import tpu as pltpu
from jax.experimental.pallas import tpu_sc as plsc

N_NODES = 10000
DEG = 32
D = 128
OUT_DIM = 128
N_CLASSES = 40

NC = 2
NS = 16
NW = NC * NS
BP = 10240
BPW = BP // NW
NB = 64
NBURST = BPW // NB
NROW = NB // 4


def _sc_sage_body(inputs_hbm, adjp_hbm, feat_hbm, self_hbm, sum_hbm,
                  iv, av, avc, sv, nbuf, sumbuf, sem, wsem):
    wid = lax.axis_index("s") * NC + lax.axis_index("c")
    base = wid * BPW
    pltpu.sync_copy(inputs_hbm.at[pl.ds(base, BPW)], iv)

    def burst(bi, carry):
        off = bi * NB
        idx = iv.at[pl.ds(off, NB)]
        c1 = pltpu.async_copy(adjp_hbm.at[idx], av, sem)
        c2 = pltpu.async_copy(feat_hbm.at[idx], sv, sem)
        c1.wait()
        c2.wait()
        wc = pltpu.async_copy(sv, self_hbm.at[pl.ds(base + off, NB)], wsem)

        for i in range(NB):
            r, c = divmod(i, 4)
            avc[r, pl.ds(32 * c, 16)] = av[i, pl.ds(0, 16)]
            avc[r, pl.ds(32 * c + 16, 16)] = av[i, pl.ds(16, 16)]

        pltpu.async_copy(feat_hbm.at[avc.at[0]], nbuf.at[0], sem)

        def pair(g, carry):
            for b in range(2):
                rr = 2 * g + b
                pltpu.make_async_copy(
                    feat_hbm.at[avc.at[0]], nbuf.at[b], sem).wait()

                @pl.when(rr + 1 < NROW)
                def _():
                    pltpu.async_copy(
                        feat_hbm.at[avc.at[rr + 1]], nbuf.at[1 - b], sem)

                for nl in range(4):
                    def red(k, accs):
                        return tuple(
                            accs[j] + nbuf[b, nl * 32 + k, pl.ds(j * 16, 16)]
                            for j in range(8)
                        )
                    accs = tuple(jnp.zeros((16,), jnp.float32) for _ in range(8))
                    accs = lax.fori_loop(0, 32, red, accs)
                    for j in range(8):
                        sumbuf[rr * 4 + nl, pl.ds(j * 16, 16)] = accs[j]
            return carry

        lax.fori_loop(0, NROW // 2, pair, 0)
        wc.wait()
        pltpu.async_copy(sumbuf, sum_hbm.at[pl.ds(base + off, NB)], wsem).wait()
        return carry

    lax.fori_loop(0, NBURST, burst, 0)


@functools.lru_cache(maxsize=1)
def _build_sc_kernel():
    mesh = plsc.VectorSubcoreMesh(core_axis_name="c", subcore_axis_name="s")
    return pl.kernel(
        _sc_sage_body,
        out_type=[
            jax.ShapeDtypeStruct((BP, D), jnp.float32),
            jax.ShapeDtypeStruct((BP, D), jnp.float32),
        ],
        mesh=mesh,
        scratch_types=[
            pltpu.VMEM((BPW,), jnp.int32),
            pltpu.VMEM((NB, 128), jnp.int32),
            pltpu.VMEM((NROW, 128), jnp.int32),
            pltpu.VMEM((NB, D), jnp.float32),
            pltpu.VMEM((2, 128, D), jnp.float32),
            pltpu.VMEM((NB, D), jnp.float32),
            pltpu.SemaphoreType.DMA,
            pltpu.SemaphoreType.DMA,
        ],
    )


_RB = 512


def _tc_body(self_ref, sum_ref, w1a_ref, w1s_ref, b1_ref, w3_ref, b3_ref, out_ref):
    x = jnp.dot(self_ref[...], w1a_ref[...], preferred_element_type=jnp.float32)
    x = x + jnp.dot(sum_ref[...], w1s_ref[...], preferred_element_type=jnp.float32)
    x = jnp.maximum(x + b1_ref[...], 0.0)
    l = jnp.dot(x, w3_ref[...], preferred_element_type=jnp.float32) + b3_ref[...]
    ss = jnp.sum(l * l, axis=1, keepdims=True)
    denom = jnp.maximum(jnp.sqrt(ss), 1e-12)
    out_ref[...] = l / denom


def kernel(inputs, adj, feat_data, W1, b1, W3, b3):
    B = inputs.shape[0]
    inputs_p = jnp.concatenate(
        [inputs.astype(jnp.int32), jnp.zeros((BP - B,), jnp.int32)])
    adj_p = jnp.pad(adj, ((0, 0), (0, 128 - DEG)))

    self_feat, sums = _build_sc_kernel()(inputs_p, adj_p, feat_data)

    w1a_t = W1[:, :D].T
    w1s_t = (W1[:, D:] * (1.0 / DEG)).T
    w3_t = jnp.pad(W3.T, ((0, 0), (0, 128 - N_CLASSES)))
    b1_r = b1.reshape(1, OUT_DIM)
    b3_r = jnp.pad(b3, (0, 128 - N_CLASSES)).reshape(1, 128)

    logits = pl.pallas_call(
        _tc_body,
        out_shape=jax.ShapeDtypeStruct((BP, 128), jnp.float32),
        grid=(BP // _RB,),
        in_specs=[
            pl.BlockSpec((_RB, D), lambda i: (i, 0)),
            pl.BlockSpec((_RB, D), lambda i: (i, 0)),
            pl.BlockSpec((D, OUT_DIM), lambda i: (0, 0)),
            pl.BlockSpec((D, OUT_DIM), lambda i: (0, 0)),
            pl.BlockSpec((1, OUT_DIM), lambda i: (0, 0)),
            pl.BlockSpec((OUT_DIM, 128), lambda i: (0, 0)),
            pl.BlockSpec((1, 128), lambda i: (0, 0)),
        ],
        out_specs=pl.BlockSpec((_RB, 128), lambda i: (i, 0)),
    )(self_feat, sums, w1a_t, w1s_t, b1_r, w3_t, b3_r)

    return logits[:B, :N_CLASSES]
